# baseline (speedup 1.0000x reference)
import functools

import jax
import jax.numpy as jnp
from jax import lax
from jax.experimental import pallas as pl
from jax.experimental.pallas import tpu as pltpu
from jax.experimental.pallas import tpu_sc as plsc

SEQ_LEN = 8192
FEAT_DIM = 2048
MAX_MISSING = 0.2

NUMEL = SEQ_LEN * FEAT_DIM
K_KEEP = int((1.0 - MAX_MISSING) * NUMEL)
RANK = NUMEL - K_KEEP

ROWS_PER_BLOCK = 512
HBINS = 65536
NWORKERS = 32
HALF_ROWS = SEQ_LEN // 2
ROWS_PER_W = HALF_ROWS // NWORKERS
CHUNK_ROWS = 8
NCHUNKS = ROWS_PER_W // CHUNK_ROWS


def _key_body(logits_ref, u_ref, out_ref):
    g = -jnp.log(-jnp.log(u_ref[...] + 1e-10))
    v = logits_ref[...] + g
    b = lax.bitcast_convert_type(v, jnp.int32)
    out_ref[...] = jnp.where(b >= 0, b, b ^ jnp.int32(0x7FFFFFFF))


_SC_MESH = plsc.VectorSubcoreMesh(core_axis_name="c", subcore_axis_name="s")
_SC_PARAMS = pltpu.CompilerParams(needs_layout_passes=False)


def _zero_hist(hist_v):
    zeros = jnp.zeros((16,), jnp.int32)

    @plsc.parallel_loop(0, 512)
    def _(i):
        for c in range(8):
            hist_v[i, pl.ds(c * 16, 16)] = zeros


def _scan_shard(keys_hbm, chunk_a, chunk_b, sem_a, sem_b, wid, scatter16):
    base = wid * ROWS_PER_W

    def process(buf):
        @plsc.parallel_loop(0, FEAT_DIM // 16, unroll=2)
        def _(i):
            for r in range(CHUNK_ROWS):
                kv = buf[r, pl.ds(i * 16, 16)]
                scatter16(kv)

    def start(ci, buf, sem):
        pltpu.async_copy(
            keys_hbm.at[pl.ds(base + ci * CHUNK_ROWS, CHUNK_ROWS)], buf, sem)

    def drain(buf, sem):
        pltpu.make_async_copy(
            keys_hbm.at[pl.ds(base, CHUNK_ROWS)], buf, sem).wait()

    start(0, chunk_a, sem_a)

    def body2(j, carry):
        ci = 2 * j

        @pl.when(ci + 1 < NCHUNKS)
        def _():
            start(ci + 1, chunk_b, sem_b)

        drain(chunk_a, sem_a)
        process(chunk_a)

        @pl.when(ci + 2 < NCHUNKS)
        def _():
            start(ci + 2, chunk_a, sem_a)

        @pl.when(ci + 1 < NCHUNKS)
        def _():
            drain(chunk_b, sem_b)
            process(chunk_b)

        return carry

    lax.fori_loop(0, (NCHUNKS + 1) // 2, body2, 0)


_HIST_SCRATCH = [
    pltpu.VMEM((CHUNK_ROWS, FEAT_DIM), jnp.int32),
    pltpu.VMEM((CHUNK_ROWS, FEAT_DIM), jnp.int32),
    pltpu.SemaphoreType.DMA,
    pltpu.SemaphoreType.DMA,
    pltpu.VMEM((512, 128), jnp.int32),
]


@functools.partial(
    pl.kernel,
    mesh=_SC_MESH,
    out_type=jax.ShapeDtypeStruct((NWORKERS, 512, 128), jnp.int32),
    scratch_types=_HIST_SCRATCH,
    compiler_params=_SC_PARAMS,
)
def _hist_hi(keys_hbm, out_hbm, chunk_a, chunk_b, sem_a, sem_b, hist_v):
    wid = lax.axis_index("s") * 2 + lax.axis_index("c")
    _zero_hist(hist_v)
    ones = jnp.ones((16,), jnp.int32)

    def scatter16(kv):
        bucket = (kv >> 16) + 32768
        plsc.addupdate_scatter(hist_v, [bucket >> 7, bucket & 127], ones)

    _scan_shard(keys_hbm, chunk_a, chunk_b, sem_a, sem_b, wid, scatter16)
    pltpu.sync_copy(hist_v, out_hbm.at[wid])


@functools.partial(
    pl.kernel,
    mesh=_SC_MESH,
    out_type=jax.ShapeDtypeStruct((NWORKERS, 512, 128), jnp.int32),
    scratch_types=_HIST_SCRATCH + [pltpu.VMEM((16,), jnp.int32)],
    compiler_params=_SC_PARAMS,
)
def _hist_lo(keys_top_hbm, keys_bot_hbm, bvec_hbm, out_hbm, chunk_a, chunk_b,
             sem_a, sem_b, hist_v, bbuf_v):
    wid = lax.axis_index("s") * 2 + lax.axis_index("c")
    _zero_hist(hist_v)
    pltpu.sync_copy(bvec_hbm, bbuf_v)
    bvec = bbuf_v[...]
    ones = jnp.ones((16,), jnp.int32)

    def scatter16(kv):
        bucket = (kv >> 16) + 32768
        low = kv & 0xFFFF
        m = bucket == bvec
        plsc.addupdate_scatter(hist_v, [low >> 7, low & 127], ones, mask=m)

    _scan_shard(keys_top_hbm, chunk_a, chunk_b, sem_a, sem_b, wid, scatter16)
    _scan_shard(keys_bot_hbm, chunk_a, chunk_b, sem_a, sem_b, wid, scatter16)
    pltpu.sync_copy(hist_v, out_hbm.at[wid])


def _cumsum_flat(h):
    hsum = jnp.sum(h, axis=0)
    i0 = lax.broadcasted_iota(jnp.int32, (128, 128), 0)
    i1 = lax.broadcasted_iota(jnp.int32, (128, 128), 1)
    upper = (i0 <= i1).astype(jnp.float32)
    row_cs = jnp.dot(hsum, upper, preferred_element_type=jnp.float32,
                     precision=lax.Precision.HIGHEST)
    j0 = lax.broadcasted_iota(jnp.int32, (512, 512), 0)
    j1 = lax.broadcasted_iota(jnp.int32, (512, 512), 1)
    strict_lower = (j1 < j0).astype(jnp.float32)
    row_tot = jnp.sum(hsum, axis=1, keepdims=True)
    prefix = jnp.dot(strict_lower, jnp.broadcast_to(row_tot, (512, 128)),
                     preferred_element_type=jnp.float32,
                     precision=lax.Precision.HIGHEST)
    return row_cs + prefix


def _select_hi_body(hist_a_ref, hist_b_ref, b_ref, rp_ref):
    cs = _cumsum_flat(hist_a_ref[...].astype(jnp.float32)
                      + hist_b_ref[...].astype(jnp.float32))
    le = cs <= float(RANK)
    bstar = jnp.sum(le.astype(jnp.float32)).astype(jnp.int32)
    below = jnp.max(jnp.where(le, cs, 0.0)).astype(jnp.int32)
    rp = RANK - below
    b_ref[...] = jnp.zeros((8, 128), jnp.int32) + bstar
    rp_ref[...] = jnp.zeros((8, 128), jnp.int32) + rp


def _select_lo_body(hist_ref, b_smem, rp_smem, t_ref):
    rp = rp_smem[0].astype(jnp.float32)
    cs = _cumsum_flat(hist_ref[...].astype(jnp.float32))
    lstar = jnp.sum((cs <= rp).astype(jnp.float32)).astype(jnp.int32)
    bstar = b_smem[0]
    t = ((bstar - 32768) << 16) | lstar
    t_ref[...] = jnp.zeros((8, 128), jnp.int32) + t


def _mask_body(top_ref, bot_ref, t_ref, out_ref):
    i = pl.program_id(0)
    t = t_ref[0]

    @pl.when(i < SEQ_LEN // ROWS_PER_BLOCK // 2)
    def _():
        out_ref[...] = (top_ref[...] >= t).astype(jnp.float32)

    @pl.when(i >= SEQ_LEN // ROWS_PER_BLOCK // 2)
    def _():
        out_ref[...] = (bot_ref[...] >= t).astype(jnp.float32)


def _key_half(logits, u, block_offset):
    grid = (HALF_ROWS // ROWS_PER_BLOCK,)
    in_spec = pl.BlockSpec((ROWS_PER_BLOCK, FEAT_DIM),
                           lambda i, o=block_offset: (i + o, 0))
    out_spec = pl.BlockSpec((ROWS_PER_BLOCK, FEAT_DIM), lambda i: (i, 0))
    return pl.pallas_call(
        _key_body,
        grid=grid,
        in_specs=[in_spec, in_spec],
        out_specs=out_spec,
        out_shape=jax.ShapeDtypeStruct((HALF_ROWS, FEAT_DIM), jnp.int32),
    )(logits, u)


def kernel(x, logits, u):
    del x
    nhalf = HALF_ROWS // ROWS_PER_BLOCK

    keys_top = _key_half(logits, u, 0)
    hist1a = _hist_hi(keys_top)
    keys_bot = _key_half(logits, u, nhalf)
    hist1b = _hist_hi(keys_bot)

    sel_b, sel_rp = pl.pallas_call(
        _select_hi_body,
        in_specs=[pl.BlockSpec((NWORKERS, 512, 128), lambda: (0, 0, 0))] * 2,
        out_specs=[pl.BlockSpec((8, 128), lambda: (0, 0))] * 2,
        out_shape=[jax.ShapeDtypeStruct((8, 128), jnp.int32)] * 2,
    )(hist1a, hist1b)

    bvec16 = sel_b[0, 0:16]
    hist2 = _hist_lo(keys_top, keys_bot, bvec16)

    tsplat = pl.pallas_call(
        _select_lo_body,
        in_specs=[
            pl.BlockSpec((NWORKERS, 512, 128), lambda: (0, 0, 0)),
            pl.BlockSpec(memory_space=pltpu.SMEM),
            pl.BlockSpec(memory_space=pltpu.SMEM),
        ],
        out_specs=pl.BlockSpec((8, 128), lambda: (0, 0)),
        out_shape=jax.ShapeDtypeStruct((8, 128), jnp.int32),
    )(hist2, sel_b[0, 0:1], sel_rp[0, 0:1])

    m_hard = pl.pallas_call(
        _mask_body,
        grid=(SEQ_LEN // ROWS_PER_BLOCK,),
        in_specs=[
            pl.BlockSpec((ROWS_PER_BLOCK, FEAT_DIM),
                         lambda i: (jnp.minimum(i, nhalf - 1), 0)),
            pl.BlockSpec((ROWS_PER_BLOCK, FEAT_DIM),
                         lambda i: (jnp.maximum(i - nhalf, 0), 0)),
            pl.BlockSpec(memory_space=pltpu.SMEM),
        ],
        out_specs=pl.BlockSpec((ROWS_PER_BLOCK, FEAT_DIM), lambda i: (i, 0)),
        out_shape=jax.ShapeDtypeStruct((SEQ_LEN, FEAT_DIM), jnp.float32),
    )(keys_top, keys_bot, tsplat[0, 0:1])
    return m_hard


# speedup vs baseline: 1.0165x; 1.0165x over previous
import functools

import jax
import jax.numpy as jnp
from jax import lax
from jax.experimental import pallas as pl
from jax.experimental.pallas import tpu as pltpu
from jax.experimental.pallas import tpu_sc as plsc

SEQ_LEN = 8192
FEAT_DIM = 2048
MAX_MISSING = 0.2

NUMEL = SEQ_LEN * FEAT_DIM
K_KEEP = int((1.0 - MAX_MISSING) * NUMEL)
RANK = NUMEL - K_KEEP

ROWS_PER_BLOCK = 512
HBINS = 65536
NWORKERS = 32
ROWS_PER_W = SEQ_LEN // NWORKERS
CHUNK_ROWS = 8
NCHUNKS = ROWS_PER_W // CHUNK_ROWS


def _key_body(logits_ref, u_ref, out_ref):
    g = -jnp.log(-jnp.log(u_ref[...] + 1e-10))
    v = logits_ref[...] + g
    b = lax.bitcast_convert_type(v, jnp.int32)
    out_ref[...] = jnp.where(b >= 0, b, b ^ jnp.int32(0x7FFFFFFF))


_SC_MESH = plsc.VectorSubcoreMesh(core_axis_name="c", subcore_axis_name="s")
_SC_PARAMS = pltpu.CompilerParams(needs_layout_passes=False)


def _zero_hist(hist_v):
    zeros = jnp.zeros((16,), jnp.int32)

    @plsc.parallel_loop(0, 512)
    def _(i):
        for c in range(8):
            hist_v[i, pl.ds(c * 16, 16)] = zeros


def _hist_pass(keys_hbm, out_hbm, chunk_a, chunk_b, sem_a, sem_b, hist_v,
               wid, scatter16):
    base = wid * ROWS_PER_W

    def process(buf):
        @plsc.parallel_loop(0, FEAT_DIM // 16, unroll=2)
        def _(i):
            for r in range(CHUNK_ROWS):
                kv = buf[r, pl.ds(i * 16, 16)]
                scatter16(kv)

    def start(ci, buf, sem):
        pltpu.async_copy(
            keys_hbm.at[pl.ds(base + ci * CHUNK_ROWS, CHUNK_ROWS)], buf, sem)

    def drain(buf, sem):
        pltpu.make_async_copy(
            keys_hbm.at[pl.ds(base, CHUNK_ROWS)], buf, sem).wait()

    start(0, chunk_a, sem_a)

    def body2(j, carry):
        ci = 2 * j

        @pl.when(ci + 1 < NCHUNKS)
        def _():
            start(ci + 1, chunk_b, sem_b)

        drain(chunk_a, sem_a)
        process(chunk_a)

        @pl.when(ci + 2 < NCHUNKS)
        def _():
            start(ci + 2, chunk_a, sem_a)

        @pl.when(ci + 1 < NCHUNKS)
        def _():
            drain(chunk_b, sem_b)
            process(chunk_b)

        return carry

    lax.fori_loop(0, (NCHUNKS + 1) // 2, body2, 0)
    pltpu.sync_copy(hist_v, out_hbm.at[wid])


_HIST_SCRATCH = [
    pltpu.VMEM((CHUNK_ROWS, FEAT_DIM), jnp.int32),
    pltpu.VMEM((CHUNK_ROWS, FEAT_DIM), jnp.int32),
    pltpu.SemaphoreType.DMA,
    pltpu.SemaphoreType.DMA,
    pltpu.VMEM((512, 128), jnp.int32),
]


@functools.partial(
    pl.kernel,
    mesh=_SC_MESH,
    out_type=jax.ShapeDtypeStruct((NWORKERS, 512, 128), jnp.int32),
    scratch_types=_HIST_SCRATCH,
    compiler_params=_SC_PARAMS,
)
def _hist_hi(keys_hbm, out_hbm, chunk_a, chunk_b, sem_a, sem_b, hist_v):
    wid = lax.axis_index("s") * 2 + lax.axis_index("c")
    _zero_hist(hist_v)
    ones = jnp.ones((16,), jnp.int32)

    def scatter16(kv):
        bucket = (kv >> 16) + 32768
        plsc.addupdate_scatter(hist_v, [bucket >> 7, bucket & 127], ones)

    _hist_pass(keys_hbm, out_hbm, chunk_a, chunk_b, sem_a, sem_b, hist_v,
               wid, scatter16)


@functools.partial(
    pl.kernel,
    mesh=_SC_MESH,
    out_type=jax.ShapeDtypeStruct((NWORKERS, 512, 128), jnp.int32),
    scratch_types=_HIST_SCRATCH + [pltpu.VMEM((16,), jnp.int32)],
    compiler_params=_SC_PARAMS,
)
def _hist_lo(keys_hbm, bvec_hbm, out_hbm, chunk_a, chunk_b, sem_a, sem_b,
             hist_v, bbuf_v):
    wid = lax.axis_index("s") * 2 + lax.axis_index("c")
    _zero_hist(hist_v)
    pltpu.sync_copy(bvec_hbm, bbuf_v)
    bvec = bbuf_v[...]
    ones = jnp.ones((16,), jnp.int32)

    def scatter16(kv):
        bucket = (kv >> 16) + 32768
        low = kv & 0xFFFF
        m = bucket == bvec
        plsc.addupdate_scatter(hist_v, [low >> 7, low & 127], ones, mask=m)

    _hist_pass(keys_hbm, out_hbm, chunk_a, chunk_b, sem_a, sem_b, hist_v,
               wid, scatter16)


def _cumsum_flat(h):
    hsum = jnp.sum(h, axis=0)
    i0 = lax.broadcasted_iota(jnp.int32, (128, 128), 0)
    i1 = lax.broadcasted_iota(jnp.int32, (128, 128), 1)
    upper = (i0 <= i1).astype(jnp.float32)
    row_cs = jnp.dot(hsum, upper, preferred_element_type=jnp.float32,
                     precision=lax.Precision.HIGHEST)
    j0 = lax.broadcasted_iota(jnp.int32, (512, 512), 0)
    j1 = lax.broadcasted_iota(jnp.int32, (512, 512), 1)
    strict_lower = (j1 < j0).astype(jnp.float32)
    row_tot = jnp.sum(hsum, axis=1, keepdims=True)
    prefix = jnp.dot(strict_lower, jnp.broadcast_to(row_tot, (512, 128)),
                     preferred_element_type=jnp.float32,
                     precision=lax.Precision.HIGHEST)
    return row_cs + prefix


def _select_hi_body(hist_ref, b_ref, rp_ref):
    cs = _cumsum_flat(hist_ref[...].astype(jnp.float32))
    le = cs <= float(RANK)
    bstar = jnp.sum(le.astype(jnp.float32)).astype(jnp.int32)
    below = jnp.max(jnp.where(le, cs, 0.0)).astype(jnp.int32)
    rp = RANK - below
    b_ref[...] = jnp.zeros((8, 128), jnp.int32) + bstar
    rp_ref[...] = jnp.zeros((8, 128), jnp.int32) + rp


def _select_lo_threshold(hist, b_smem, rp_smem):
    rp = rp_smem[0].astype(jnp.float32)
    cs = _cumsum_flat(hist.astype(jnp.float32))
    lstar = jnp.sum((cs <= rp).astype(jnp.float32)).astype(jnp.int32)
    bstar = b_smem[0]
    return ((bstar - 32768) << 16) | lstar


def _mask_body(keys_ref, hist_ref, b_smem, rp_smem, out_ref, t_s):
    @pl.when(pl.program_id(0) == 0)
    def _():
        t_s[0] = _select_lo_threshold(hist_ref[...], b_smem, rp_smem)

    out_ref[...] = (keys_ref[...] >= t_s[0]).astype(jnp.float32)


def kernel(x, logits, u):
    del x
    grid = (SEQ_LEN // ROWS_PER_BLOCK,)
    bspec = pl.BlockSpec((ROWS_PER_BLOCK, FEAT_DIM), lambda i: (i, 0))

    keys2d = pl.pallas_call(
        _key_body,
        grid=grid,
        in_specs=[bspec, bspec],
        out_specs=bspec,
        out_shape=jax.ShapeDtypeStruct((SEQ_LEN, FEAT_DIM), jnp.int32),
    )(logits, u)

    hist1 = _hist_hi(keys2d)

    sel_b, sel_rp = pl.pallas_call(
        _select_hi_body,
        in_specs=[pl.BlockSpec((NWORKERS, 512, 128), lambda: (0, 0, 0))],
        out_specs=[pl.BlockSpec((8, 128), lambda: (0, 0))] * 2,
        out_shape=[jax.ShapeDtypeStruct((8, 128), jnp.int32)] * 2,
    )(hist1)

    bvec16 = sel_b[0, 0:16]
    hist2 = _hist_lo(keys2d, bvec16)

    m_hard = pl.pallas_call(
        _mask_body,
        grid=grid,
        in_specs=[
            bspec,
            pl.BlockSpec((NWORKERS, 512, 128), lambda i: (0, 0, 0)),
            pl.BlockSpec(memory_space=pltpu.SMEM),
            pl.BlockSpec(memory_space=pltpu.SMEM),
        ],
        out_specs=bspec,
        out_shape=jax.ShapeDtypeStruct((SEQ_LEN, FEAT_DIM), jnp.float32),
        scratch_shapes=[pltpu.SMEM((1,), jnp.int32)],
    )(keys2d, hist2, sel_b[0, 0:1], sel_rp[0, 0:1])
    return m_hard


# speedup vs baseline: 1.0209x; 1.0043x over previous
import functools

import jax
import jax.numpy as jnp
from jax import lax
from jax.experimental import pallas as pl
from jax.experimental.pallas import tpu as pltpu
from jax.experimental.pallas import tpu_sc as plsc

SEQ_LEN = 8192
FEAT_DIM = 2048
MAX_MISSING = 0.2

NUMEL = SEQ_LEN * FEAT_DIM
K_KEEP = int((1.0 - MAX_MISSING) * NUMEL)
RANK = NUMEL - K_KEEP

ROWS_PER_BLOCK = 512
HBINS = 65536
NWORKERS = 32
ROWS_PER_W = SEQ_LEN // NWORKERS
CHUNK_ROWS = 8
NCHUNKS = ROWS_PER_W // CHUNK_ROWS


def _key_body(logits_ref, u_ref, out_ref):
    g = -jnp.log(-jnp.log(u_ref[...] + 1e-10))
    v = logits_ref[...] + g
    b = lax.bitcast_convert_type(v, jnp.int32)
    out_ref[...] = jnp.where(b >= 0, b, b ^ jnp.int32(0x7FFFFFFF))


_SC_MESH = plsc.VectorSubcoreMesh(core_axis_name="c", subcore_axis_name="s")
_SC_PARAMS = pltpu.CompilerParams(needs_layout_passes=False)


def _zero_hist(hist_v):
    zeros = jnp.zeros((16,), jnp.int32)

    @plsc.parallel_loop(0, 512)
    def _(i):
        for c in range(8):
            hist_v[i, pl.ds(c * 16, 16)] = zeros


def _hist_pass(keys_hbm, out_hbm, chunk_a, chunk_b, sem_a, sem_b, hist_v,
               wid, scatter16):
    base = wid * ROWS_PER_W

    def process(buf):
        @plsc.parallel_loop(0, FEAT_DIM // 16, unroll=4)
        def _(i):
            for r in range(CHUNK_ROWS):
                kv = buf[r, pl.ds(i * 16, 16)]
                scatter16(kv)

    def start(ci, buf, sem):
        pltpu.async_copy(
            keys_hbm.at[pl.ds(base + ci * CHUNK_ROWS, CHUNK_ROWS)], buf, sem)

    def drain(buf, sem):
        pltpu.make_async_copy(
            keys_hbm.at[pl.ds(base, CHUNK_ROWS)], buf, sem).wait()

    start(0, chunk_a, sem_a)

    def body2(j, carry):
        ci = 2 * j

        @pl.when(ci + 1 < NCHUNKS)
        def _():
            start(ci + 1, chunk_b, sem_b)

        drain(chunk_a, sem_a)
        process(chunk_a)

        @pl.when(ci + 2 < NCHUNKS)
        def _():
            start(ci + 2, chunk_a, sem_a)

        @pl.when(ci + 1 < NCHUNKS)
        def _():
            drain(chunk_b, sem_b)
            process(chunk_b)

        return carry

    lax.fori_loop(0, (NCHUNKS + 1) // 2, body2, 0)
    pltpu.sync_copy(hist_v, out_hbm.at[wid])


_HIST_SCRATCH = [
    pltpu.VMEM((CHUNK_ROWS, FEAT_DIM), jnp.int32),
    pltpu.VMEM((CHUNK_ROWS, FEAT_DIM), jnp.int32),
    pltpu.SemaphoreType.DMA,
    pltpu.SemaphoreType.DMA,
    pltpu.VMEM((512, 128), jnp.int32),
]


@functools.partial(
    pl.kernel,
    mesh=_SC_MESH,
    out_type=jax.ShapeDtypeStruct((NWORKERS, 512, 128), jnp.int32),
    scratch_types=_HIST_SCRATCH,
    compiler_params=_SC_PARAMS,
)
def _hist_hi(keys_hbm, out_hbm, chunk_a, chunk_b, sem_a, sem_b, hist_v):
    wid = lax.axis_index("s") * 2 + lax.axis_index("c")
    _zero_hist(hist_v)
    ones = jnp.ones((16,), jnp.int32)

    def scatter16(kv):
        bucket = (kv >> 16) + 32768
        plsc.addupdate_scatter(hist_v, [bucket >> 7, bucket & 127], ones)

    _hist_pass(keys_hbm, out_hbm, chunk_a, chunk_b, sem_a, sem_b, hist_v,
               wid, scatter16)


@functools.partial(
    pl.kernel,
    mesh=_SC_MESH,
    out_type=jax.ShapeDtypeStruct((NWORKERS, 512, 128), jnp.int32),
    scratch_types=_HIST_SCRATCH + [pltpu.VMEM((16,), jnp.int32)],
    compiler_params=_SC_PARAMS,
)
def _hist_lo(keys_hbm, bvec_hbm, out_hbm, chunk_a, chunk_b, sem_a, sem_b,
             hist_v, bbuf_v):
    wid = lax.axis_index("s") * 2 + lax.axis_index("c")
    _zero_hist(hist_v)
    pltpu.sync_copy(bvec_hbm, bbuf_v)
    bvec = bbuf_v[...]
    ones = jnp.ones((16,), jnp.int32)

    def scatter16(kv):
        bucket = (kv >> 16) + 32768
        low = kv & 0xFFFF
        m = bucket == bvec
        plsc.addupdate_scatter(hist_v, [low >> 7, low & 127], ones, mask=m)

    _hist_pass(keys_hbm, out_hbm, chunk_a, chunk_b, sem_a, sem_b, hist_v,
               wid, scatter16)


def _cumsum_flat(h):
    hsum = jnp.sum(h, axis=0)
    i0 = lax.broadcasted_iota(jnp.int32, (128, 128), 0)
    i1 = lax.broadcasted_iota(jnp.int32, (128, 128), 1)
    upper = (i0 <= i1).astype(jnp.float32)
    row_cs = jnp.dot(hsum, upper, preferred_element_type=jnp.float32,
                     precision=lax.Precision.HIGHEST)
    j0 = lax.broadcasted_iota(jnp.int32, (512, 512), 0)
    j1 = lax.broadcasted_iota(jnp.int32, (512, 512), 1)
    strict_lower = (j1 < j0).astype(jnp.float32)
    row_tot = jnp.sum(hsum, axis=1, keepdims=True)
    prefix = jnp.dot(strict_lower, jnp.broadcast_to(row_tot, (512, 128)),
                     preferred_element_type=jnp.float32,
                     precision=lax.Precision.HIGHEST)
    return row_cs + prefix


def _select_hi_body(hist_ref, b_ref, rp_ref):
    cs = _cumsum_flat(hist_ref[...].astype(jnp.float32))
    le = cs <= float(RANK)
    bstar = jnp.sum(le.astype(jnp.float32)).astype(jnp.int32)
    below = jnp.max(jnp.where(le, cs, 0.0)).astype(jnp.int32)
    rp = RANK - below
    b_ref[...] = jnp.zeros((8, 128), jnp.int32) + bstar
    rp_ref[...] = jnp.zeros((8, 128), jnp.int32) + rp


def _select_lo_threshold(hist, b_smem, rp_smem):
    rp = rp_smem[0].astype(jnp.float32)
    cs = _cumsum_flat(hist.astype(jnp.float32))
    lstar = jnp.sum((cs <= rp).astype(jnp.float32)).astype(jnp.int32)
    bstar = b_smem[0]
    return ((bstar - 32768) << 16) | lstar


def _mask_body(keys_ref, hist_ref, b_smem, rp_smem, out_ref, t_s):
    @pl.when(pl.program_id(0) == 0)
    def _():
        t_s[0] = _select_lo_threshold(hist_ref[...], b_smem, rp_smem)

    out_ref[...] = (keys_ref[...] >= t_s[0]).astype(jnp.float32)


def kernel(x, logits, u):
    del x
    grid = (SEQ_LEN // ROWS_PER_BLOCK,)
    bspec = pl.BlockSpec((ROWS_PER_BLOCK, FEAT_DIM), lambda i: (i, 0))

    keys2d = pl.pallas_call(
        _key_body,
        grid=grid,
        in_specs=[bspec, bspec],
        out_specs=bspec,
        out_shape=jax.ShapeDtypeStruct((SEQ_LEN, FEAT_DIM), jnp.int32),
    )(logits, u)

    hist1 = _hist_hi(keys2d)

    sel_b, sel_rp = pl.pallas_call(
        _select_hi_body,
        in_specs=[pl.BlockSpec((NWORKERS, 512, 128), lambda: (0, 0, 0))],
        out_specs=[pl.BlockSpec((8, 128), lambda: (0, 0))] * 2,
        out_shape=[jax.ShapeDtypeStruct((8, 128), jnp.int32)] * 2,
    )(hist1)

    bvec16 = sel_b[0, 0:16]
    hist2 = _hist_lo(keys2d, bvec16)

    m_hard = pl.pallas_call(
        _mask_body,
        grid=grid,
        in_specs=[
            bspec,
            pl.BlockSpec((NWORKERS, 512, 128), lambda i: (0, 0, 0)),
            pl.BlockSpec(memory_space=pltpu.SMEM),
            pl.BlockSpec(memory_space=pltpu.SMEM),
        ],
        out_specs=bspec,
        out_shape=jax.ShapeDtypeStruct((SEQ_LEN, FEAT_DIM), jnp.float32),
        scratch_shapes=[pltpu.SMEM((1,), jnp.int32)],
    )(keys2d, hist2, sel_b[0, 0:1], sel_rp[0, 0:1])
    return m_hard
